# baseline (speedup 1.0000x reference)
import functools

import jax
import jax.numpy as jnp
from jax import lax
from jax.experimental import pallas as pl
from jax.experimental.pallas import tpu as pltpu
from jax.experimental.pallas import tpu_sc as plsc

B = 16384
L = 32
H = 512
NC = 2
NS = 16
NW = NC * NS
BPW = B // NW
GB = 2
RG = GB * L
GROUPS = BPW // GB
IDX_ROWS = BPW * L // RG


def _tree_sum(vs):
    while len(vs) > 1:
        vs = [vs[i] + vs[i + 1] for i in range(0, len(vs) - 1, 2)] + (
            [vs[-1]] if len(vs) % 2 else [])
    return vs[0]


def _reduce_group(rows_v, acc_v):
    def col_body(c, _2):
        for cc in range(2):
            co = c * 32 + cc * 16
            for s in range(GB):
                acc = _tree_sum([rows_v[s * L + j, pl.ds(co, 16)]
                                 for j in range(L)])
                acc_v[pl.ds(s * H + co, 16)] = acc
        return 0

    lax.fori_loop(0, H // 32, col_body, 0)


def _bag_body(x_hbm, table_hbm, out_hbm, idx_v, rows0, rows1, acc0, acc1,
              gsem0, gsem1, osem0, osem1):
    wid = lax.axis_index("s") * NC + lax.axis_index("c")
    base = wid * BPW
    rows = (rows0, rows1)
    accs = (acc0, acc1)
    gsems = (gsem0, gsem1)
    osems = (osem0, osem1)

    pltpu.sync_copy(x_hbm.at[pl.ds(wid * IDX_ROWS, IDX_ROWS)], idx_v)

    pltpu.async_copy(table_hbm.at[idx_v.at[0]], rows0, gsem0)
    pltpu.async_copy(table_hbm.at[idx_v.at[1]], rows1, gsem1)

    def pair_body(p, _):
        for b in range(2):
            g = p * 2 + b
            pltpu.make_async_copy(table_hbm.at[idx_v.at[g]], rows[b],
                                  gsems[b]).wait()
            @pl.when(p > 0)
            def _wait_out():
                pltpu.make_async_copy(
                    accs[b],
                    out_hbm.at[pl.ds((base + (g - 2) * GB) * H, GB * H)],
                    osems[b]).wait()

            _reduce_group(rows[b], accs[b])

            @pl.when(g + 2 < GROUPS)
            def _next_gather():
                pltpu.async_copy(table_hbm.at[idx_v.at[g + 2]], rows[b],
                                 gsems[b])

            pltpu.async_copy(
                accs[b],
                out_hbm.at[pl.ds((base + g * GB) * H, GB * H)],
                osems[b])
        return 0

    lax.fori_loop(0, GROUPS // 2, pair_body, 0)

    for b in range(2):
        g = GROUPS - 2 + b
        pltpu.make_async_copy(
            accs[b],
            out_hbm.at[pl.ds((base + g * GB) * H, GB * H)],
            osems[b]).wait()


@functools.partial(
    pl.kernel,
    mesh=plsc.VectorSubcoreMesh(core_axis_name="c", subcore_axis_name="s"),
    out_type=jax.ShapeDtypeStruct((B * H,), jnp.float32),
    scratch_types=[
        pltpu.VMEM((IDX_ROWS, RG), jnp.int32),
        pltpu.VMEM((RG, H), jnp.float32),
        pltpu.VMEM((RG, H), jnp.float32),
        pltpu.VMEM((GB * H,), jnp.float32),
        pltpu.VMEM((GB * H,), jnp.float32),
        pltpu.SemaphoreType.DMA,
        pltpu.SemaphoreType.DMA,
        pltpu.SemaphoreType.DMA,
        pltpu.SemaphoreType.DMA,
    ],
)
def _bag_kernel(x_hbm, table_hbm, out_hbm, idx_v, rows0, rows1, acc0, acc1,
                gsem0, gsem1, osem0, osem1):
    _bag_body(x_hbm, table_hbm, out_hbm, idx_v, rows0, rows1, acc0, acc1,
              gsem0, gsem1, osem0, osem1)


BB = 1024


def _head_body(bag_ref, b1_ref, w2t_ref, b2_ref, wht_ref, bh_ref, out_ref):
    h = jnp.maximum(bag_ref[...] + b1_ref[...], 0.0)
    h2 = jnp.dot(h, w2t_ref[...], preferred_element_type=jnp.float32)
    h2 = jnp.maximum(h2 + b2_ref[...], 0.0)
    out_ref[...] = (
        jnp.dot(h2, wht_ref[...], preferred_element_type=jnp.float32)
        + bh_ref[...]
    )


def _head_call(bag, b1, w2t, b2, wht, bh):
    return pl.pallas_call(
        _head_body,
        grid=(B // BB,),
        in_specs=[
            pl.BlockSpec((BB, H), lambda i: (i, 0)),
            pl.BlockSpec((H,), lambda i: (0,)),
            pl.BlockSpec((H, 32), lambda i: (0, 0)),
            pl.BlockSpec((32,), lambda i: (0,)),
            pl.BlockSpec((32, 4), lambda i: (0, 0)),
            pl.BlockSpec((4,), lambda i: (0,)),
        ],
        out_specs=pl.BlockSpec((BB, 4), lambda i: (i, 0)),
        out_shape=jax.ShapeDtypeStruct((B, 4), jnp.float32),
    )(bag, b1, w2t, b2, wht, bh)


def kernel(x, table, bias1, W2, b2, Wcp, bcp, Wwdl, bwdl):
    x_idx = x.reshape(B * L // RG, RG)
    bag = _bag_kernel(x_idx, table).reshape(B, H)
    wht = jnp.concatenate([Wcp, Wwdl], axis=0).T
    bh = jnp.concatenate([bcp, bwdl], axis=0)
    out = _head_call(bag, bias1, W2.T, b2, wht, bh)
    return out[:, :1], out[:, 1:4]


# speedup vs baseline: 1.5348x; 1.5348x over previous
import functools

import jax
import jax.numpy as jnp
from jax import lax
from jax.experimental import pallas as pl
from jax.experimental.pallas import tpu as pltpu
from jax.experimental.pallas import tpu_sc as plsc

B = 16384
L = 32
H = 512
HW = H // 2
NC = 2
NS = 16
NW = NC * NS
BPW = B // NW
GB = 4
RG = GB * L
GROUPS = BPW // GB
IDX_ROWS = BPW * L // RG


def _tree_sum(vs):
    while len(vs) > 1:
        vs = [vs[i] + vs[i + 1] for i in range(0, len(vs) - 1, 2)] + (
            [vs[-1]] if len(vs) % 2 else [])
    return vs[0]


def _reduce_group(rows_v, acc_v):
    def col_body(c, _2):
        co = c * 16
        for s in range(GB):
            def l_body(l8, accs):
                aa, ab = accs
                r = s * L + l8 * 8
                was, wbs = [], []
                for j in range(8):
                    w = rows_v[r + j, pl.ds(co, 16)]
                    was.append(lax.bitcast_convert_type(
                        lax.shift_left(w, 16), jnp.float32))
                    wbs.append(lax.bitcast_convert_type(
                        jnp.bitwise_and(w, jnp.int32(-65536)), jnp.float32))
                return aa + _tree_sum(was), ab + _tree_sum(wbs)

            zero = jnp.zeros((16,), jnp.float32)
            aa, ab = lax.fori_loop(0, L // 8, l_body, (zero, zero))
            acc_v[pl.ds(s * H + co, 16)] = aa
            acc_v[pl.ds(s * H + HW + co, 16)] = ab
        return 0

    lax.fori_loop(0, HW // 16, col_body, 0)


def _bag_body(x_hbm, table_hbm, out_hbm, idx_v, rows0, rows1, rows2,
              acc0, acc1, acc2, gsem0, gsem1, gsem2, osem0, osem1, osem2):
    wid = lax.axis_index("s") * NC + lax.axis_index("c")
    base = wid * BPW
    rows = (rows0, rows1, rows2)
    accs = (acc0, acc1, acc2)
    gsems = (gsem0, gsem1, gsem2)
    osems = (osem0, osem1, osem2)

    pltpu.sync_copy(x_hbm.at[pl.ds(wid * IDX_ROWS, IDX_ROWS)], idx_v)

    pltpu.async_copy(table_hbm.at[idx_v.at[0]], rows0, gsem0)
    pltpu.async_copy(table_hbm.at[idx_v.at[1]], rows1, gsem1)

    def trip_body(p, _):
        for b in range(3):
            g = p * 3 + b

            @pl.when(g < GROUPS)
            def _grp():
                pltpu.make_async_copy(table_hbm.at[idx_v.at[g]], rows[b],
                                      gsems[b]).wait()

                @pl.when(g + 2 < GROUPS)
                def _next_gather():
                    pltpu.async_copy(
                        table_hbm.at[idx_v.at[g + 2]],
                        rows[(b + 2) % 3], gsems[(b + 2) % 3])

                @pl.when(p > 0)
                def _wait_out():
                    pltpu.make_async_copy(
                        accs[b],
                        out_hbm.at[pl.ds((base + (g - 3) * GB) * H,
                                         GB * H)],
                        osems[b]).wait()

                _reduce_group(rows[b], accs[b])

                pltpu.async_copy(
                    accs[b],
                    out_hbm.at[pl.ds((base + g * GB) * H, GB * H)],
                    osems[b])
        return 0

    lax.fori_loop(0, (GROUPS + 2) // 3, trip_body, 0)

    for g in range(GROUPS - 3, GROUPS):
        b = g % 3
        pltpu.make_async_copy(
            accs[b],
            out_hbm.at[pl.ds((base + g * GB) * H, GB * H)],
            osems[b]).wait()


@functools.partial(
    pl.kernel,
    mesh=plsc.VectorSubcoreMesh(core_axis_name="c", subcore_axis_name="s"),
    out_type=jax.ShapeDtypeStruct((B * H,), jnp.float32),
    scratch_types=[
        pltpu.VMEM((IDX_ROWS, RG), jnp.int32),
        pltpu.VMEM((RG, HW), jnp.int32),
        pltpu.VMEM((RG, HW), jnp.int32),
        pltpu.VMEM((RG, HW), jnp.int32),
        pltpu.VMEM((GB * H,), jnp.float32),
        pltpu.VMEM((GB * H,), jnp.float32),
        pltpu.VMEM((GB * H,), jnp.float32),
        pltpu.SemaphoreType.DMA,
        pltpu.SemaphoreType.DMA,
        pltpu.SemaphoreType.DMA,
        pltpu.SemaphoreType.DMA,
        pltpu.SemaphoreType.DMA,
        pltpu.SemaphoreType.DMA,
    ],
)
def _bag_kernel(x_hbm, table_hbm, out_hbm, idx_v, rows0, rows1, rows2,
                acc0, acc1, acc2, gsem0, gsem1, gsem2, osem0, osem1, osem2):
    _bag_body(x_hbm, table_hbm, out_hbm, idx_v, rows0, rows1, rows2,
              acc0, acc1, acc2, gsem0, gsem1, gsem2, osem0, osem1, osem2)


BB = 1024


def _head_body(bag_ref, b1_ref, w2t_ref, b2_ref, wht_ref, bh_ref, out_ref):
    h = jnp.maximum(bag_ref[...] + b1_ref[...], 0.0)
    h2 = jnp.dot(h, w2t_ref[...], preferred_element_type=jnp.float32)
    h2 = jnp.maximum(h2 + b2_ref[...], 0.0)
    out_ref[...] = (
        jnp.dot(h2, wht_ref[...], preferred_element_type=jnp.float32)
        + bh_ref[...]
    )


def _head_call(bag, b1, w2t, b2, wht, bh):
    return pl.pallas_call(
        _head_body,
        grid=(B // BB,),
        in_specs=[
            pl.BlockSpec((BB, H), lambda i: (i, 0)),
            pl.BlockSpec((H,), lambda i: (0,)),
            pl.BlockSpec((H, 32), lambda i: (0, 0)),
            pl.BlockSpec((32,), lambda i: (0,)),
            pl.BlockSpec((32, 4), lambda i: (0, 0)),
            pl.BlockSpec((4,), lambda i: (0,)),
        ],
        out_specs=pl.BlockSpec((BB, 4), lambda i: (i, 0)),
        out_shape=jax.ShapeDtypeStruct((B, 4), jnp.float32),
    )(bag, b1, w2t, b2, wht, bh)


def kernel(x, table, bias1, W2, b2, Wcp, bcp, Wwdl, bwdl):
    x_idx = x.reshape(B * L // RG, RG)
    tb = table.astype(jnp.bfloat16)
    lo = jax.lax.bitcast_convert_type(tb[:, :HW], jnp.uint16).astype(
        jnp.uint32)
    hi = jax.lax.bitcast_convert_type(tb[:, HW:], jnp.uint16).astype(
        jnp.uint32)
    table_i32 = jax.lax.bitcast_convert_type(
        lo | (hi << jnp.uint32(16)), jnp.int32)
    bag = _bag_kernel(x_idx, table_i32).reshape(B, H)
    wht = jnp.concatenate([Wcp, Wwdl], axis=0).T
    bh = jnp.concatenate([bcp, bwdl], axis=0)
    out = _head_call(bag, bias1, W2.T, b2, wht, bh)
    return out[:, :1], out[:, 1:4]


# speedup vs baseline: 1.5974x; 1.0408x over previous
import functools

import jax
import jax.numpy as jnp
from jax import lax
from jax.experimental import pallas as pl
from jax.experimental.pallas import tpu as pltpu
from jax.experimental.pallas import tpu_sc as plsc

B = 16384
L = 32
H = 512
HW = H // 2
NC = 2
NS = 16
NW = NC * NS
BPW = B // NW
GB = 4
RG = GB * L
GROUPS = BPW // GB
IDX_ROWS = BPW * L // RG


def _tree_sum(vs):
    while len(vs) > 1:
        vs = [vs[i] + vs[i + 1] for i in range(0, len(vs) - 1, 2)] + (
            [vs[-1]] if len(vs) % 2 else [])
    return vs[0]


def _reduce_group(rows_v, acc_v):
    def col_body(c, _2):
        co = c * 16
        for s in range(GB):
            def l_body(l8, accs):
                aa, ab = accs
                r = s * L + l8 * 8
                was, wbs = [], []
                for j in range(8):
                    w = rows_v[r + j, pl.ds(co, 16)]
                    was.append(lax.bitcast_convert_type(
                        lax.shift_left(w, 16), jnp.float32))
                    wbs.append(lax.bitcast_convert_type(
                        jnp.bitwise_and(w, jnp.int32(-65536)), jnp.float32))
                return aa + _tree_sum(was), ab + _tree_sum(wbs)

            zero = jnp.zeros((16,), jnp.float32)
            aa, ab = lax.fori_loop(0, L // 8, l_body, (zero, zero))
            acc_v[pl.ds(s * H + co, 16)] = aa
            acc_v[pl.ds(s * H + HW + co, 16)] = ab
        return 0

    lax.fori_loop(0, HW // 16, col_body, 0)


def _bag_body(x_hbm, table_hbm, out_hbm, idx_v, rows0, rows1, rows2,
              acc0, acc1, acc2, gsem0, gsem1, gsem2, osem0, osem1, osem2):
    wid = lax.axis_index("s") * NC + lax.axis_index("c")
    base = wid * BPW
    rows = (rows0, rows1, rows2)
    accs = (acc0, acc1, acc2)
    gsems = (gsem0, gsem1, gsem2)
    osems = (osem0, osem1, osem2)

    pltpu.sync_copy(x_hbm.at[pl.ds(wid * IDX_ROWS, IDX_ROWS)], idx_v)

    pltpu.async_copy(table_hbm.at[idx_v.at[0]], rows0, gsem0)
    pltpu.async_copy(table_hbm.at[idx_v.at[1]], rows1, gsem1)

    def trip_body(p, _):
        for b in range(3):
            g = p * 3 + b

            @pl.when(g < GROUPS)
            def _grp():
                pltpu.make_async_copy(table_hbm.at[idx_v.at[g]], rows[b],
                                      gsems[b]).wait()

                @pl.when(g + 2 < GROUPS)
                def _next_gather():
                    pltpu.async_copy(
                        table_hbm.at[idx_v.at[g + 2]],
                        rows[(b + 2) % 3], gsems[(b + 2) % 3])

                @pl.when(p > 0)
                def _wait_out():
                    pltpu.make_async_copy(
                        accs[b],
                        out_hbm.at[pl.ds((base + (g - 3) * GB) * H,
                                         GB * H)],
                        osems[b]).wait()

                _reduce_group(rows[b], accs[b])

                pltpu.async_copy(
                    accs[b],
                    out_hbm.at[pl.ds((base + g * GB) * H, GB * H)],
                    osems[b])
        return 0

    lax.fori_loop(0, (GROUPS + 2) // 3, trip_body, 0)

    for g in range(GROUPS - 3, GROUPS):
        b = g % 3
        pltpu.make_async_copy(
            accs[b],
            out_hbm.at[pl.ds((base + g * GB) * H, GB * H)],
            osems[b]).wait()


@functools.partial(
    pl.kernel,
    mesh=plsc.VectorSubcoreMesh(core_axis_name="c", subcore_axis_name="s"),
    out_type=jax.ShapeDtypeStruct((B * H,), jnp.float32),
    scratch_types=[
        pltpu.VMEM((IDX_ROWS, RG), jnp.int32),
        pltpu.VMEM((RG, HW), jnp.int32),
        pltpu.VMEM((RG, HW), jnp.int32),
        pltpu.VMEM((RG, HW), jnp.int32),
        pltpu.VMEM((GB * H,), jnp.float32),
        pltpu.VMEM((GB * H,), jnp.float32),
        pltpu.VMEM((GB * H,), jnp.float32),
        pltpu.SemaphoreType.DMA,
        pltpu.SemaphoreType.DMA,
        pltpu.SemaphoreType.DMA,
        pltpu.SemaphoreType.DMA,
        pltpu.SemaphoreType.DMA,
        pltpu.SemaphoreType.DMA,
    ],
)
def _bag_kernel(x_hbm, table_hbm, out_hbm, idx_v, rows0, rows1, rows2,
                acc0, acc1, acc2, gsem0, gsem1, gsem2, osem0, osem1, osem2):
    _bag_body(x_hbm, table_hbm, out_hbm, idx_v, rows0, rows1, rows2,
              acc0, acc1, acc2, gsem0, gsem1, gsem2, osem0, osem1, osem2)


PB = 1024
NROWS = 40961


def _pack_body(t_ref, out_ref):
    xl = t_ref[:, :HW].astype(jnp.bfloat16)
    xh = t_ref[:, HW:].astype(jnp.bfloat16)
    lo = jax.lax.bitcast_convert_type(xl, jnp.uint16).astype(jnp.uint32)
    hi = jax.lax.bitcast_convert_type(xh, jnp.uint16).astype(jnp.uint32)
    out_ref[...] = jax.lax.bitcast_convert_type(
        lo | (hi << jnp.uint32(16)), jnp.int32)


def _pack_call(table):
    return pl.pallas_call(
        _pack_body,
        grid=((NROWS + PB - 1) // PB,),
        in_specs=[pl.BlockSpec((PB, H), lambda i: (i, 0))],
        out_specs=pl.BlockSpec((PB, HW), lambda i: (i, 0)),
        out_shape=jax.ShapeDtypeStruct((NROWS, HW), jnp.int32),
    )(table)


BB = 1024


def _head_body(bag_ref, b1_ref, w2t_ref, b2_ref, wht_ref, bh_ref, out_ref):
    h = jnp.maximum(bag_ref[...] + b1_ref[...], 0.0)
    h2 = jnp.dot(h, w2t_ref[...], preferred_element_type=jnp.float32)
    h2 = jnp.maximum(h2 + b2_ref[...], 0.0)
    out_ref[...] = (
        jnp.dot(h2, wht_ref[...], preferred_element_type=jnp.float32)
        + bh_ref[...]
    )


def _head_call(bag, b1, w2t, b2, wht, bh):
    return pl.pallas_call(
        _head_body,
        grid=(B // BB,),
        in_specs=[
            pl.BlockSpec((BB, H), lambda i: (i, 0)),
            pl.BlockSpec((H,), lambda i: (0,)),
            pl.BlockSpec((H, 32), lambda i: (0, 0)),
            pl.BlockSpec((32,), lambda i: (0,)),
            pl.BlockSpec((32, 4), lambda i: (0, 0)),
            pl.BlockSpec((4,), lambda i: (0,)),
        ],
        out_specs=pl.BlockSpec((BB, 4), lambda i: (i, 0)),
        out_shape=jax.ShapeDtypeStruct((B, 4), jnp.float32),
    )(bag, b1, w2t, b2, wht, bh)


def kernel(x, table, bias1, W2, b2, Wcp, bcp, Wwdl, bwdl):
    x_idx = x.reshape(B * L // RG, RG)
    table_i32 = _pack_call(table)
    bag = _bag_kernel(x_idx, table_i32).reshape(B, H)
    wht = jnp.concatenate([Wcp, Wwdl], axis=0).T
    bh = jnp.concatenate([bcp, bwdl], axis=0)
    out = _head_call(bag, bias1, W2.T, b2, wht, bh)
    return out[:, :1], out[:, 1:4]


# speedup vs baseline: 1.6110x; 1.0085x over previous
import functools

import jax
import jax.numpy as jnp
from jax import lax
from jax.experimental import pallas as pl
from jax.experimental.pallas import tpu as pltpu
from jax.experimental.pallas import tpu_sc as plsc

B = 16384
L = 32
H = 512
HW = H // 2
NC = 2
NS = 16
NW = NC * NS
BPW = B // NW
GB = 4
RG = GB * L
GROUPS = BPW // GB
IDX_ROWS = BPW * L // RG


def _tree_sum(vs):
    while len(vs) > 1:
        vs = [vs[i] + vs[i + 1] for i in range(0, len(vs) - 1, 2)] + (
            [vs[-1]] if len(vs) % 2 else [])
    return vs[0]


def _reduce_group(rows_v, acc_v):
    def col_body(c, _2):
        co = c * 16
        for s in range(GB):
            def l_body(l8, accs):
                aa, ab = accs
                r = s * L + l8 * 8
                was, wbs = [], []
                for j in range(8):
                    w = rows_v[r + j, pl.ds(co, 16)]
                    was.append(lax.bitcast_convert_type(
                        lax.shift_left(w, 16), jnp.float32))
                    wbs.append(lax.bitcast_convert_type(
                        jnp.bitwise_and(w, jnp.int32(-65536)), jnp.float32))
                return aa + _tree_sum(was), ab + _tree_sum(wbs)

            zero = jnp.zeros((16,), jnp.float32)
            aa, ab = lax.fori_loop(0, L // 8, l_body, (zero, zero))
            acc_v[pl.ds(s * H + co, 16)] = aa
            acc_v[pl.ds(s * H + HW + co, 16)] = ab
        return 0

    lax.fori_loop(0, HW // 16, col_body, 0)


def _bag_body(x_hbm, table_hbm, out_hbm, idx_v, rows0, rows1, rows2,
              acc0, acc1, acc2, gsem0, gsem1, gsem2, osem0, osem1, osem2):
    wid = lax.axis_index("s") * NC + lax.axis_index("c")
    base = wid * BPW
    rows = (rows0, rows1, rows2)
    accs = (acc0, acc1, acc2)
    gsems = (gsem0, gsem1, gsem2)
    osems = (osem0, osem1, osem2)

    pltpu.sync_copy(x_hbm.at[pl.ds(wid * IDX_ROWS, IDX_ROWS)], idx_v)

    pltpu.async_copy(table_hbm.at[idx_v.at[0]], rows0, gsem0)
    pltpu.async_copy(table_hbm.at[idx_v.at[1]], rows1, gsem1)

    def trip_body(p, _):
        for b in range(3):
            g = p * 3 + b

            @pl.when(g < GROUPS)
            def _grp():
                pltpu.make_async_copy(table_hbm.at[idx_v.at[g]], rows[b],
                                      gsems[b]).wait()

                @pl.when(g + 2 < GROUPS)
                def _next_gather():
                    pltpu.async_copy(
                        table_hbm.at[idx_v.at[g + 2]],
                        rows[(b + 2) % 3], gsems[(b + 2) % 3])

                @pl.when(p > 0)
                def _wait_out():
                    pltpu.make_async_copy(
                        accs[b],
                        out_hbm.at[pl.ds((base + (g - 3) * GB) * H,
                                         GB * H)],
                        osems[b]).wait()

                _reduce_group(rows[b], accs[b])

                pltpu.async_copy(
                    accs[b],
                    out_hbm.at[pl.ds((base + g * GB) * H, GB * H)],
                    osems[b])
        return 0

    lax.fori_loop(0, (GROUPS + 2) // 3, trip_body, 0)

    for g in range(GROUPS - 3, GROUPS):
        b = g % 3
        pltpu.make_async_copy(
            accs[b],
            out_hbm.at[pl.ds((base + g * GB) * H, GB * H)],
            osems[b]).wait()


@functools.partial(
    pl.kernel,
    mesh=plsc.VectorSubcoreMesh(core_axis_name="c", subcore_axis_name="s"),
    out_type=jax.ShapeDtypeStruct((B * H,), jnp.float32),
    scratch_types=[
        pltpu.VMEM((IDX_ROWS, RG), jnp.int32),
        pltpu.VMEM((RG, HW), jnp.int32),
        pltpu.VMEM((RG, HW), jnp.int32),
        pltpu.VMEM((RG, HW), jnp.int32),
        pltpu.VMEM((GB * H,), jnp.float32),
        pltpu.VMEM((GB * H,), jnp.float32),
        pltpu.VMEM((GB * H,), jnp.float32),
        pltpu.SemaphoreType.DMA,
        pltpu.SemaphoreType.DMA,
        pltpu.SemaphoreType.DMA,
        pltpu.SemaphoreType.DMA,
        pltpu.SemaphoreType.DMA,
        pltpu.SemaphoreType.DMA,
    ],
)
def _bag_kernel(x_hbm, table_hbm, out_hbm, idx_v, rows0, rows1, rows2,
                acc0, acc1, acc2, gsem0, gsem1, gsem2, osem0, osem1, osem2):
    _bag_body(x_hbm, table_hbm, out_hbm, idx_v, rows0, rows1, rows2,
              acc0, acc1, acc2, gsem0, gsem1, gsem2, osem0, osem1, osem2)


PB = 1024
NROWS = 40961


def _pack_body(t_ref, out_ref):
    xl = t_ref[:, :HW].astype(jnp.bfloat16)
    xh = t_ref[:, HW:].astype(jnp.bfloat16)
    lo = jax.lax.bitcast_convert_type(xl, jnp.uint16).astype(jnp.uint32)
    hi = jax.lax.bitcast_convert_type(xh, jnp.uint16).astype(jnp.uint32)
    out_ref[...] = jax.lax.bitcast_convert_type(
        lo | (hi << jnp.uint32(16)), jnp.int32)


def _pack_call(table):
    return pl.pallas_call(
        _pack_body,
        grid=((NROWS + PB - 1) // PB,),
        in_specs=[pl.BlockSpec((PB, H), lambda i: (i, 0))],
        out_specs=pl.BlockSpec((PB, HW), lambda i: (i, 0)),
        out_shape=jax.ShapeDtypeStruct((NROWS, HW), jnp.int32),
    )(table)


BB = 1024


def _head_body(bag_ref, b1_ref, w2t_ref, b2_ref, wcpt_ref, bcp_ref,
               wwdlt_ref, bwdl_ref, cp_ref, wdl_ref):
    h = jnp.maximum(bag_ref[...] + b1_ref[...], 0.0)
    h2 = jnp.dot(h, w2t_ref[...], preferred_element_type=jnp.float32)
    h2 = jnp.maximum(h2 + b2_ref[...], 0.0)
    cp_ref[...] = (
        jnp.dot(h2, wcpt_ref[...], preferred_element_type=jnp.float32)
        + bcp_ref[...]
    )
    wdl_ref[...] = (
        jnp.dot(h2, wwdlt_ref[...], preferred_element_type=jnp.float32)
        + bwdl_ref[...]
    )


def _head_call(bag, b1, w2t, b2, wcpt, bcp, wwdlt, bwdl):
    return pl.pallas_call(
        _head_body,
        grid=(B // BB,),
        in_specs=[
            pl.BlockSpec((BB, H), lambda i: (i, 0)),
            pl.BlockSpec((H,), lambda i: (0,)),
            pl.BlockSpec((H, 32), lambda i: (0, 0)),
            pl.BlockSpec((32,), lambda i: (0,)),
            pl.BlockSpec((32, 1), lambda i: (0, 0)),
            pl.BlockSpec((1,), lambda i: (0,)),
            pl.BlockSpec((32, 3), lambda i: (0, 0)),
            pl.BlockSpec((3,), lambda i: (0,)),
        ],
        out_specs=[
            pl.BlockSpec((BB, 1), lambda i: (i, 0)),
            pl.BlockSpec((BB, 3), lambda i: (i, 0)),
        ],
        out_shape=[
            jax.ShapeDtypeStruct((B, 1), jnp.float32),
            jax.ShapeDtypeStruct((B, 3), jnp.float32),
        ],
    )(bag, b1, w2t, b2, wcpt, bcp, wwdlt, bwdl)


def kernel(x, table, bias1, W2, b2, Wcp, bcp, Wwdl, bwdl):
    x_idx = x.reshape(B * L // RG, RG)
    table_i32 = _pack_call(table)
    bag = _bag_kernel(x_idx, table_i32).reshape(B, H)
    cp, wdl = _head_call(bag, bias1, W2.T, b2, Wcp.T, bcp, Wwdl.T, bwdl)
    return cp, wdl
